# baseline (speedup 1.0000x reference)
import functools

import jax
import jax.numpy as jnp
from jax import lax
from jax.experimental import pallas as pl
from jax.experimental.pallas import tpu as pltpu
from jax.experimental.pallas import tpu_sc as plsc

_NBUF = 5
_NPACK = 2
_VCH = 16384


def _widen_table(table_t):
    D, V = table_t.shape

    def body(t_ref, o_ref):
        o_ref[:, 0:D] = t_ref[...].T

    return pl.pallas_call(
        body,
        grid=(pl.cdiv(V, _VCH),),
        in_specs=[pl.BlockSpec((D, _VCH), lambda i: (0, i))],
        out_specs=pl.BlockSpec((_VCH, 128), lambda i: (i, 0)),
        out_shape=jax.ShapeDtypeStruct((V, 128), table_t.dtype),
    )(table_t)


def kernel(inputs, table):
    B, T = inputs.shape
    V, D = table.shape
    idx_t = inputs.T.astype(jnp.int32)
    table_ext = _widen_table(table.T)

    info = plsc.get_sparse_core_info()
    nw = info.num_cores * info.num_subcores
    bw = B // nw
    ng = bw // 16

    mesh = plsc.VectorSubcoreMesh(core_axis_name="c", subcore_axis_name="s")

    @functools.partial(
        pl.kernel,
        out_type=jax.ShapeDtypeStruct((T, D, B), table.dtype),
        mesh=mesh,
        compiler_params=pltpu.CompilerParams(needs_layout_passes=False),
        scratch_types=[
            pltpu.VMEM((T, bw), jnp.int32),
            pltpu.VMEM((_NBUF, bw, 128), jnp.float32),
            pltpu.VMEM((_NPACK, D, bw), jnp.float32),
            pltpu.SemaphoreType.DMA,
            pltpu.SemaphoreType.DMA,
            pltpu.SemaphoreType.DMA,
        ],
    )
    def gather_kernel(
        tab_hbm, idx_hbm, out_hbm, idx_v, rows_v, pack_v, gsem, osem0, osem1
    ):
        osems = (osem0, osem1)
        wid = lax.axis_index("s") * info.num_cores + lax.axis_index("c")
        b0 = wid * bw
        pltpu.sync_copy(idx_hbm.at[:, pl.ds(b0, bw)], idx_v)

        lane = lax.iota(jnp.int32, 16)

        for k in range(_NBUF):
            pltpu.make_async_copy(
                tab_hbm.at[idx_v.at[k]], rows_v.at[k], gsem
            ).start()

        @pl.loop(0, T, step=_NBUF)
        def _(t):
            for k in range(_NBUF):
                kp = k % _NPACK
                pltpu.make_async_copy(
                    tab_hbm.at[idx_v.at[t + k]], rows_v.at[k], gsem
                ).wait()

                def drain():
                    pltpu.make_async_copy(
                        pack_v.at[kp], out_hbm.at[t + k, :, pl.ds(b0, bw)],
                        osems[kp],
                    ).wait()

                if k < _NPACK:
                    pl.when(t > 0)(drain)
                else:
                    drain()

                @pl.loop(0, ng)
                def _(g):
                    bvec = lane + 16 * g
                    for s0 in range(0, D, 16):
                        dvecs = [
                            ((lane + s0 + i) & 15) + (16 * ((s0 + i) // 16))
                            for i in range(16)
                        ]
                        xs = [
                            plsc.load_gather(rows_v.at[k], [bvec, dv])
                            for dv in dvecs
                        ]
                        for dv, x in zip(dvecs, xs):
                            plsc.store_scatter(pack_v.at[kp], [dv, bvec], x)

                pltpu.make_async_copy(
                    pack_v.at[kp], out_hbm.at[t + k, :, pl.ds(b0, bw)],
                    osems[kp],
                ).start()

                @pl.when(t + k + _NBUF < T)
                def _():
                    pltpu.make_async_copy(
                        tab_hbm.at[idx_v.at[t + k + _NBUF]], rows_v.at[k],
                        gsem,
                    ).start()

        for kp in range(_NPACK):
            pltpu.make_async_copy(
                pack_v.at[kp], out_hbm.at[0, :, pl.ds(b0, bw)], osems[kp]
            ).wait()

    return gather_kernel(table_ext, idx_t).transpose(2, 0, 1)


# speedup vs baseline: 1.0338x; 1.0338x over previous
import functools

import jax
import jax.numpy as jnp
from jax import lax
from jax.experimental import pallas as pl
from jax.experimental.pallas import tpu as pltpu
from jax.experimental.pallas import tpu_sc as plsc

_NBUF = 5
_NPACK = 3
_VCH = 16384


def _widen_table(table_t):
    D, V = table_t.shape

    def body(t_ref, o_ref):
        o_ref[:, 0:D] = t_ref[...].T

    return pl.pallas_call(
        body,
        grid=(pl.cdiv(V, _VCH),),
        in_specs=[pl.BlockSpec((D, _VCH), lambda i: (0, i))],
        out_specs=pl.BlockSpec((_VCH, 128), lambda i: (i, 0)),
        out_shape=jax.ShapeDtypeStruct((V, 128), table_t.dtype),
    )(table_t)


def kernel(inputs, table):
    B, T = inputs.shape
    V, D = table.shape
    idx_t = inputs.T.astype(jnp.int32)
    table_ext = _widen_table(table.T)

    info = plsc.get_sparse_core_info()
    nw = info.num_cores * info.num_subcores
    bw = B // nw
    ng = bw // 16

    mesh = plsc.VectorSubcoreMesh(core_axis_name="c", subcore_axis_name="s")

    @functools.partial(
        pl.kernel,
        out_type=jax.ShapeDtypeStruct((T, D, B), table.dtype),
        mesh=mesh,
        compiler_params=pltpu.CompilerParams(needs_layout_passes=False),
        scratch_types=[
            pltpu.VMEM((T, bw), jnp.int32),
            pltpu.VMEM((_NBUF, bw, 128), jnp.float32),
            pltpu.VMEM((_NPACK, D, bw), jnp.float32),
            pltpu.SemaphoreType.DMA,
            pltpu.SemaphoreType.DMA,
            pltpu.SemaphoreType.DMA,
            pltpu.SemaphoreType.DMA,
        ],
    )
    def gather_kernel(
        tab_hbm, idx_hbm, out_hbm, idx_v, rows_v, pack_v, gsem,
        osem0, osem1, osem2,
    ):
        osems = (osem0, osem1, osem2)
        wid = lax.axis_index("s") * info.num_cores + lax.axis_index("c")
        b0 = wid * bw
        pltpu.sync_copy(idx_hbm.at[:, pl.ds(b0, bw)], idx_v)

        lane = lax.iota(jnp.int32, 16)

        for k in range(_NBUF):
            pltpu.make_async_copy(
                tab_hbm.at[idx_v.at[k]], rows_v.at[k], gsem
            ).start()

        @pl.loop(0, T, step=_NBUF)
        def _(t):
            for k in range(_NBUF):
                kp = k % _NPACK
                pltpu.make_async_copy(
                    tab_hbm.at[idx_v.at[t + k]], rows_v.at[k], gsem
                ).wait()

                def drain():
                    pltpu.make_async_copy(
                        pack_v.at[kp], out_hbm.at[t + k, :, pl.ds(b0, bw)],
                        osems[kp],
                    ).wait()

                if k < _NPACK:
                    pl.when(t > 0)(drain)
                else:
                    drain()

                @pl.loop(0, ng)
                def _(g):
                    bvec = lane + 16 * g
                    for s0 in range(0, D, 8):
                        dvecs = [
                            ((lane + s0 + i) & 15) + (16 * ((s0 + i) // 16))
                            for i in range(8)
                        ]
                        xs = [
                            plsc.load_gather(rows_v.at[k], [bvec, dv])
                            for dv in dvecs
                        ]
                        for dv, x in zip(dvecs, xs):
                            plsc.store_scatter(pack_v.at[kp], [dv, bvec], x)

                pltpu.make_async_copy(
                    pack_v.at[kp], out_hbm.at[t + k, :, pl.ds(b0, bw)],
                    osems[kp],
                ).start()

                @pl.when(t + k + _NBUF < T)
                def _():
                    pltpu.make_async_copy(
                        tab_hbm.at[idx_v.at[t + k + _NBUF]], rows_v.at[k],
                        gsem,
                    ).start()

        for kp in range(_NPACK):
            pltpu.make_async_copy(
                pack_v.at[kp], out_hbm.at[0, :, pl.ds(b0, bw)], osems[kp]
            ).wait()

    return gather_kernel(table_ext, idx_t).transpose(2, 0, 1)


# speedup vs baseline: 1.0527x; 1.0183x over previous
import functools

import jax
import jax.numpy as jnp
from jax import lax
from jax.experimental import pallas as pl
from jax.experimental.pallas import tpu as pltpu
from jax.experimental.pallas import tpu_sc as plsc

_NBUF = 5
_NPACK = 2
_VCH = 16384


def _widen_table(table_t):
    D, V = table_t.shape

    def body(t_ref, o_ref):
        o_ref[:, 0:D] = t_ref[...].T

    return pl.pallas_call(
        body,
        grid=(pl.cdiv(V, _VCH),),
        in_specs=[pl.BlockSpec((D, _VCH), lambda i: (0, i))],
        out_specs=pl.BlockSpec((_VCH, 128), lambda i: (i, 0)),
        out_shape=jax.ShapeDtypeStruct((V, 128), table_t.dtype),
    )(table_t)


def kernel(inputs, table):
    B, T = inputs.shape
    V, D = table.shape
    idx_t = inputs.T.astype(jnp.int32)
    table_ext = _widen_table(table.T)

    info = plsc.get_sparse_core_info()
    nw = info.num_cores * info.num_subcores
    bw = B // nw
    ng = bw // 16

    mesh = plsc.VectorSubcoreMesh(core_axis_name="c", subcore_axis_name="s")

    @functools.partial(
        pl.kernel,
        out_type=jax.ShapeDtypeStruct((T, D, B), table.dtype),
        mesh=mesh,
        compiler_params=pltpu.CompilerParams(needs_layout_passes=False),
        scratch_types=[
            pltpu.VMEM((T, bw), jnp.int32),
            pltpu.VMEM((_NBUF, bw, 128), jnp.float32),
            pltpu.VMEM((_NPACK, D, bw), jnp.float32),
            pltpu.SemaphoreType.DMA,
            pltpu.SemaphoreType.DMA,
            pltpu.SemaphoreType.DMA,
        ],
    )
    def gather_kernel(
        tab_hbm, idx_hbm, out_hbm, idx_v, rows_v, pack_v, gsem, osem0, osem1
    ):
        osems = (osem0, osem1)
        wid = lax.axis_index("s") * info.num_cores + lax.axis_index("c")
        b0 = wid * bw
        pltpu.sync_copy(idx_hbm.at[:, pl.ds(b0, bw)], idx_v)

        lane = lax.iota(jnp.int32, 16)

        for k in range(_NBUF):
            pltpu.make_async_copy(
                tab_hbm.at[idx_v.at[k]], rows_v.at[k], gsem
            ).start()

        @pl.loop(0, T, step=_NBUF)
        def _(t):
            for k in range(_NBUF):
                kp = k % _NPACK
                pltpu.make_async_copy(
                    tab_hbm.at[idx_v.at[t + k]], rows_v.at[k], gsem
                ).wait()

                def drain():
                    pltpu.make_async_copy(
                        pack_v.at[kp], out_hbm.at[t + k, :, pl.ds(b0, bw)],
                        osems[kp],
                    ).wait()

                if k < _NPACK:
                    pl.when(t > 0)(drain)
                else:
                    drain()

                @pl.loop(0, ng)
                def _(g):
                    bvec = lane + 16 * g
                    for s0 in range(0, D, 8):
                        dvecs = [
                            ((lane + s0 + i) & 15) + (16 * ((s0 + i) // 16))
                            for i in range(8)
                        ]
                        xs = [
                            plsc.load_gather(rows_v.at[k], [bvec, dv])
                            for dv in dvecs
                        ]
                        for dv, x in zip(dvecs, xs):
                            plsc.store_scatter(pack_v.at[kp], [dv, bvec], x)

                pltpu.make_async_copy(
                    pack_v.at[kp], out_hbm.at[t + k, :, pl.ds(b0, bw)],
                    osems[kp],
                ).start()

                @pl.when(t + k + _NBUF < T)
                def _():
                    pltpu.make_async_copy(
                        tab_hbm.at[idx_v.at[t + k + _NBUF]], rows_v.at[k],
                        gsem,
                    ).start()

        for kp in range(_NPACK):
            pltpu.make_async_copy(
                pack_v.at[kp], out_hbm.at[0, :, pl.ds(b0, bw)], osems[kp]
            ).wait()

    return gather_kernel(table_ext, idx_t).transpose(2, 0, 1)
